# Initial kernel scaffold; baseline (speedup 1.0000x reference)
import jax
import jax.numpy as jnp
from jax.experimental import pallas as pl


def kernel(x, W_sel, Y):
    raise NotImplementedError("write your pallas kernel here")


# speedup vs baseline: 10.6064x; 10.6064x over previous
import jax
import jax.numpy as jnp
from jax.experimental import pallas as pl

NIN = 1024
NOUT = 1024
DEPTH = 10
NNODES = 1023
NPAD = 1024
BLK = 256


def _body(x_ref, w_ref, ytab_ref, o_ref):
    xb = x_ref[...]
    lam = jnp.dot(xb, w_ref[...], preferred_element_type=jnp.float32)
    node_iota = jax.lax.broadcasted_iota(jnp.int32, (1, NPAD), 1)
    cur = jnp.zeros((BLK, 1), jnp.int32)
    S = jnp.zeros((BLK, NPAD), jnp.float32)
    for i in range(DEPTH):
        lam_i = lam[:, i:i + 1]
        S = jnp.where(node_iota == cur, lam_i, S)
        cur = cur * 2 + 1 + (lam_i > 0).astype(jnp.int32)
    o_ref[...] = jnp.dot(S, ytab_ref[...], preferred_element_type=jnp.float32)


def kernel(x, W_sel, Y):
    orig_shape = x.shape
    x2 = x.reshape(-1, NIN) if x.ndim == 3 else x
    nb = x2.shape[0]
    wT = jnp.zeros((NIN, 16), jnp.float32).at[:, :DEPTH].set(W_sel.T)
    ytab = jnp.concatenate([Y, jnp.zeros((1, NOUT), Y.dtype)], axis=0)

    y = pl.pallas_call(
        _body,
        grid=(nb // BLK,),
        in_specs=[
            pl.BlockSpec((BLK, NIN), lambda i: (i, 0)),
            pl.BlockSpec((NIN, 16), lambda i: (0, 0)),
            pl.BlockSpec((NPAD, NOUT), lambda i: (0, 0)),
        ],
        out_specs=pl.BlockSpec((BLK, NOUT), lambda i: (i, 0)),
        out_shape=jax.ShapeDtypeStruct((nb, NOUT), jnp.float32),
    )(x2, wT, ytab)

    if orig_shape[1] != NIN:
        y = y.reshape(orig_shape[0], orig_shape[1], NOUT)
    return y


# speedup vs baseline: 10.9023x; 1.0279x over previous
import jax
import jax.numpy as jnp
from jax.experimental import pallas as pl

NIN = 1024
NOUT = 1024
DEPTH = 10
NNODES = 1023
NPAD = 1024
BLK = 256


def _body(x_ref, w_ref, ytab_ref, o_ref):
    xb = x_ref[...]
    lam = jnp.dot(xb, w_ref[...], preferred_element_type=jnp.float32)
    node_iota = jax.lax.broadcasted_iota(jnp.int32, (1, NPAD), 1)
    cur = jnp.zeros((BLK, 1), jnp.int32)
    S = jnp.zeros((BLK, NPAD), jnp.float32)
    for i in range(DEPTH):
        lam_i = lam[:, i:i + 1]
        S = jnp.where(node_iota == cur, lam_i, S)
        cur = cur * 2 + 1 + (lam_i > 0).astype(jnp.int32)
    o_ref[...] = jnp.dot(S.astype(jnp.bfloat16), ytab_ref[...],
                         preferred_element_type=jnp.float32)


def kernel(x, W_sel, Y):
    orig_shape = x.shape
    x2 = x.reshape(-1, NIN) if x.ndim == 3 else x
    nb = x2.shape[0]
    wT = jnp.zeros((NIN, 16), jnp.float32).at[:, :DEPTH].set(W_sel.T)
    ytab = jnp.concatenate([Y, jnp.zeros((1, NOUT), Y.dtype)],
                           axis=0).astype(jnp.bfloat16)

    y = pl.pallas_call(
        _body,
        grid=(nb // BLK,),
        in_specs=[
            pl.BlockSpec((BLK, NIN), lambda i: (i, 0)),
            pl.BlockSpec((NIN, 16), lambda i: (0, 0)),
            pl.BlockSpec((NPAD, NOUT), lambda i: (0, 0)),
        ],
        out_specs=pl.BlockSpec((BLK, NOUT), lambda i: (i, 0)),
        out_shape=jax.ShapeDtypeStruct((nb, NOUT), jnp.float32),
    )(x2, wT, ytab)

    if orig_shape[1] != NIN:
        y = y.reshape(orig_shape[0], orig_shape[1], NOUT)
    return y


# speedup vs baseline: 14.0466x; 1.2884x over previous
import numpy as np
import jax
import jax.numpy as jnp
from jax.experimental import pallas as pl

NIN = 1024
NOUT = 1024
DEPTH = 10
NNODES = 1023
NPAD = 1024
BLK = 256

_n = np.arange(NPAD)
_d = np.where(_n < NNODES, np.floor(np.log2(_n + 1)).astype(np.int32), 0)
_SH = np.where(_n < NNODES, DEPTH - _d, 0).astype(np.int32)
_R = np.where(_n < NNODES, _n + 1 - (1 << _d), -1).astype(np.int32)
_E = np.zeros((16, NPAD), np.float32)
_E[_d[:NNODES], _n[:NNODES]] = 1.0
_PW = np.zeros((1, 16), np.int32)
_PW[0, :DEPTH] = 1 << (DEPTH - 1 - np.arange(DEPTH))


def _body(x_ref, w_ref, e_ref, sh_ref, r_ref, ytab_ref, o_ref):
    xb = x_ref[...]
    lam = jnp.dot(xb, w_ref[...], preferred_element_type=jnp.float32)
    bits = (lam > 0).astype(jnp.int32)
    col = jax.lax.broadcasted_iota(jnp.int32, (1, 16), 1)
    pw = jnp.where(col < DEPTH, jax.lax.shift_right_logical(512, col), 0)
    p = jnp.sum(bits * pw, axis=1, keepdims=True)
    lsel = jnp.dot(lam, e_ref[...], preferred_element_type=jnp.float32)
    t = jax.lax.shift_right_logical(p, sh_ref[...])
    S = jnp.where(t == r_ref[...], lsel, 0.0).astype(jnp.bfloat16)
    o_ref[...] = jnp.dot(S, ytab_ref[...], preferred_element_type=jnp.float32)


def kernel(x, W_sel, Y):
    orig_shape = x.shape
    x2 = x.reshape(-1, NIN) if x.ndim == 3 else x
    nb = x2.shape[0]
    wT = jnp.zeros((NIN, 16), jnp.float32).at[:, :DEPTH].set(W_sel.T)
    ytab = jnp.concatenate([Y, jnp.zeros((1, NOUT), Y.dtype)],
                           axis=0).astype(jnp.bfloat16)
    sh = jnp.asarray(_SH).reshape(1, NPAD)
    r = jnp.asarray(_R).reshape(1, NPAD)
    e = jnp.asarray(_E)

    y = pl.pallas_call(
        _body,
        grid=(nb // BLK,),
        in_specs=[
            pl.BlockSpec((BLK, NIN), lambda i: (i, 0)),
            pl.BlockSpec((NIN, 16), lambda i: (0, 0)),
            pl.BlockSpec((16, NPAD), lambda i: (0, 0)),
            pl.BlockSpec((1, NPAD), lambda i: (0, 0)),
            pl.BlockSpec((1, NPAD), lambda i: (0, 0)),
            pl.BlockSpec((NPAD, NOUT), lambda i: (0, 0)),
        ],
        out_specs=pl.BlockSpec((BLK, NOUT), lambda i: (i, 0)),
        out_shape=jax.ShapeDtypeStruct((nb, NOUT), jnp.float32),
    )(x2, wT, e, sh, r, ytab)

    if orig_shape[1] != NIN:
        y = y.reshape(orig_shape[0], orig_shape[1], NOUT)
    return y


# speedup vs baseline: 16.7642x; 1.1935x over previous
import numpy as np
import jax
import jax.numpy as jnp
from jax.experimental import pallas as pl

NIN = 1024
NOUT = 1024
DEPTH = 10
NNODES = 1023
NPAD = 1024
BLK = 512

_n = np.arange(NPAD)
_d = np.where(_n < NNODES, np.floor(np.log2(_n + 1)).astype(np.int32), 0)
_SH = np.where(_n < NNODES, DEPTH - _d, 0).astype(np.int32)
_R = np.where(_n < NNODES, _n + 1 - (1 << _d), -1).astype(np.int32)
_E = np.zeros((16, NPAD), np.float32)
_E[_d[:NNODES], _n[:NNODES]] = 1.0
_PW = np.zeros((1, 16), np.int32)
_PW[0, :DEPTH] = 1 << (DEPTH - 1 - np.arange(DEPTH))


def _body(x_ref, w_ref, e_ref, sh_ref, r_ref, ytab_ref, o_ref):
    xb = x_ref[...]
    lam = jnp.dot(xb, w_ref[...], preferred_element_type=jnp.float32)
    bits = (lam > 0).astype(jnp.int32)
    col = jax.lax.broadcasted_iota(jnp.int32, (1, 16), 1)
    pw = jnp.where(col < DEPTH, jax.lax.shift_right_logical(512, col), 0)
    p = jnp.sum(bits * pw, axis=1, keepdims=True)
    lsel = jnp.dot(lam, e_ref[...], preferred_element_type=jnp.float32)
    t = jax.lax.shift_right_logical(p, sh_ref[...])
    S = jnp.where(t == r_ref[...], lsel, 0.0).astype(jnp.bfloat16)
    o_ref[...] = jnp.dot(S, ytab_ref[...], preferred_element_type=jnp.float32)


def kernel(x, W_sel, Y):
    orig_shape = x.shape
    x2 = x.reshape(-1, NIN) if x.ndim == 3 else x
    nb = x2.shape[0]
    wT = jnp.zeros((NIN, 16), jnp.float32).at[:, :DEPTH].set(W_sel.T)
    ytab = jnp.concatenate([Y, jnp.zeros((1, NOUT), Y.dtype)],
                           axis=0).astype(jnp.bfloat16)
    sh = jnp.asarray(_SH).reshape(1, NPAD)
    r = jnp.asarray(_R).reshape(1, NPAD)
    e = jnp.asarray(_E)

    y = pl.pallas_call(
        _body,
        grid=(nb // BLK,),
        in_specs=[
            pl.BlockSpec((BLK, NIN), lambda i: (i, 0)),
            pl.BlockSpec((NIN, 16), lambda i: (0, 0)),
            pl.BlockSpec((16, NPAD), lambda i: (0, 0)),
            pl.BlockSpec((1, NPAD), lambda i: (0, 0)),
            pl.BlockSpec((1, NPAD), lambda i: (0, 0)),
            pl.BlockSpec((NPAD, NOUT), lambda i: (0, 0)),
        ],
        out_specs=pl.BlockSpec((BLK, NOUT), lambda i: (i, 0)),
        out_shape=jax.ShapeDtypeStruct((nb, NOUT), jnp.float32),
    )(x2, wT, e, sh, r, ytab)

    if orig_shape[1] != NIN:
        y = y.reshape(orig_shape[0], orig_shape[1], NOUT)
    return y


# speedup vs baseline: 17.3403x; 1.0344x over previous
import numpy as np
import jax
import jax.numpy as jnp
from jax.experimental import pallas as pl

NIN = 1024
NOUT = 1024
DEPTH = 10
NNODES = 1023
NPAD = 1024
BLK = 1024

_n = np.arange(NPAD)
_d = np.where(_n < NNODES, np.floor(np.log2(_n + 1)).astype(np.int32), 0)
_SH = np.where(_n < NNODES, DEPTH - _d, 0).astype(np.int32)
_R = np.where(_n < NNODES, _n + 1 - (1 << _d), -1).astype(np.int32)
_E = np.zeros((16, NPAD), np.float32)
_E[_d[:NNODES], _n[:NNODES]] = 1.0
_PW = np.zeros((1, 16), np.int32)
_PW[0, :DEPTH] = 1 << (DEPTH - 1 - np.arange(DEPTH))


def _body(x_ref, w_ref, e_ref, sh_ref, r_ref, ytab_ref, o_ref):
    xb = x_ref[...]
    lam = jnp.dot(xb, w_ref[...], preferred_element_type=jnp.float32)
    bits = (lam > 0).astype(jnp.int32)
    col = jax.lax.broadcasted_iota(jnp.int32, (1, 16), 1)
    pw = jnp.where(col < DEPTH, jax.lax.shift_right_logical(512, col), 0)
    p = jnp.sum(bits * pw, axis=1, keepdims=True)
    lsel = jnp.dot(lam, e_ref[...], preferred_element_type=jnp.float32)
    t = jax.lax.shift_right_logical(p, sh_ref[...])
    S = jnp.where(t == r_ref[...], lsel, 0.0).astype(jnp.bfloat16)
    o_ref[...] = jnp.dot(S, ytab_ref[...], preferred_element_type=jnp.float32)


def kernel(x, W_sel, Y):
    orig_shape = x.shape
    x2 = x.reshape(-1, NIN) if x.ndim == 3 else x
    nb = x2.shape[0]
    wT = jnp.zeros((NIN, 16), jnp.float32).at[:, :DEPTH].set(W_sel.T)
    ytab = jnp.concatenate([Y, jnp.zeros((1, NOUT), Y.dtype)],
                           axis=0).astype(jnp.bfloat16)
    sh = jnp.asarray(_SH).reshape(1, NPAD)
    r = jnp.asarray(_R).reshape(1, NPAD)
    e = jnp.asarray(_E)

    y = pl.pallas_call(
        _body,
        grid=(nb // BLK,),
        in_specs=[
            pl.BlockSpec((BLK, NIN), lambda i: (i, 0)),
            pl.BlockSpec((NIN, 16), lambda i: (0, 0)),
            pl.BlockSpec((16, NPAD), lambda i: (0, 0)),
            pl.BlockSpec((1, NPAD), lambda i: (0, 0)),
            pl.BlockSpec((1, NPAD), lambda i: (0, 0)),
            pl.BlockSpec((NPAD, NOUT), lambda i: (0, 0)),
        ],
        out_specs=pl.BlockSpec((BLK, NOUT), lambda i: (i, 0)),
        out_shape=jax.ShapeDtypeStruct((nb, NOUT), jnp.float32),
    )(x2, wT, e, sh, r, ytab)

    if orig_shape[1] != NIN:
        y = y.reshape(orig_shape[0], orig_shape[1], NOUT)
    return y
